# baseline (speedup 1.0000x reference)
import functools

import jax
import jax.numpy as jnp
from jax import lax
from jax.experimental import pallas as pl
from jax.experimental.pallas import tpu as pltpu
from jax.experimental.pallas import tpu_sc as plsc

B = 16384
D = 64
NC = 2
NS = 16
NW = NC * NS
BPW = B // NW


def _sc_entity_ids(i_ids, item_map):
    mesh = plsc.VectorSubcoreMesh(core_axis_name="c", subcore_axis_name="s")

    @functools.partial(
        pl.kernel,
        mesh=mesh,
        out_type=jax.ShapeDtypeStruct((B,), jnp.int32),
        compiler_params=pltpu.CompilerParams(use_tc_tiling_on_sc=False),
        scratch_types=[
            pltpu.VMEM((BPW,), jnp.int32),
            pltpu.VMEM((BPW,), jnp.int32),
            pltpu.SemaphoreType.DMA,
        ],
    )
    def ka(i_ids_h, map_h, evar_out, iidx, evar_v, s0):
        wid = lax.axis_index("s") * NC + lax.axis_index("c")
        base = wid * BPW
        pltpu.sync_copy(i_ids_h.at[pl.ds(base, BPW)], iidx)
        pltpu.async_copy(map_h.at[iidx], evar_v, s0).wait()
        pltpu.sync_copy(evar_v, evar_out.at[pl.ds(base, BPW)])

    return ka(i_ids, item_map)


def _sc_gather_rows(u_ids, i_ids, evar, user_emb, item_emb, ent_emb):
    mesh = plsc.VectorSubcoreMesh(core_axis_name="c", subcore_axis_name="s")
    out_types = (
        jax.ShapeDtypeStruct((B, D), jnp.float32),
        jax.ShapeDtypeStruct((B, D), jnp.float32),
        jax.ShapeDtypeStruct((B, D), jnp.float32),
    )

    @functools.partial(
        pl.kernel,
        mesh=mesh,
        out_type=out_types,
        scratch_types=[
            pltpu.VMEM((BPW,), jnp.int32),
            pltpu.VMEM((BPW,), jnp.int32),
            pltpu.VMEM((BPW,), jnp.int32),
            pltpu.SemaphoreType.DMA,
            pltpu.SemaphoreType.DMA,
            pltpu.SemaphoreType.DMA,
        ],
    )
    def kb(u_ids_h, i_ids_h, evar_h, ue_h, ie_h, ee_h, u_out, i_out, e_out,
           uidx, iidx, eidx, su, si, se):
        wid = lax.axis_index("s") * NC + lax.axis_index("c")
        base = wid * BPW
        pltpu.sync_copy(u_ids_h.at[pl.ds(base, BPW)], uidx)
        pltpu.sync_copy(i_ids_h.at[pl.ds(base, BPW)], iidx)
        pltpu.sync_copy(evar_h.at[pl.ds(base, BPW)], eidx)

        @pl.loop(0, BPW, step=16)
        def _(r0):
            uvec = uidx[pl.ds(r0, 16)]
            ivec = iidx[pl.ds(r0, 16)]
            evec = eidx[pl.ds(r0, 16)]
            for j in range(16):
                pltpu.async_copy(ue_h.at[pl.ds(uvec[j], 1)],
                                 u_out.at[pl.ds(base + r0 + j, 1)], su)
                pltpu.async_copy(ie_h.at[pl.ds(ivec[j], 1)],
                                 i_out.at[pl.ds(base + r0 + j, 1)], si)
                pltpu.async_copy(ee_h.at[pl.ds(evec[j], 1)],
                                 e_out.at[pl.ds(base + r0 + j, 1)], se)

        pltpu.make_async_copy(ue_h.at[pl.ds(0, BPW)],
                              u_out.at[pl.ds(base, BPW)], su).wait()
        pltpu.make_async_copy(ie_h.at[pl.ds(0, BPW)],
                              i_out.at[pl.ds(base, BPW)], si).wait()
        pltpu.make_async_copy(ee_h.at[pl.ds(0, BPW)],
                              e_out.at[pl.ds(base, BPW)], se).wait()

    return kb(u_ids, i_ids, evar, user_emb, item_emb, ent_emb)


def _tc_score(u_rows, i_rows, e_rows):
    def body(u_ref, i_ref, e_ref, o_ref):
        o_ref[...] = jnp.sum(u_ref[...] * (i_ref[...] + e_ref[...]), axis=-1)

    return pl.pallas_call(
        body,
        out_shape=jax.ShapeDtypeStruct((B,), jnp.float32),
    )(u_rows, i_rows, e_rows)


def kernel(u_ids, i_ids, user_emb, item_emb, ent_emb, item_map):
    u_ids = u_ids.astype(jnp.int32)
    i_ids = i_ids.astype(jnp.int32)
    evar = _sc_entity_ids(i_ids, item_map.astype(jnp.int32))
    u_rows, i_rows, e_rows = _sc_gather_rows(
        u_ids, i_ids, evar, user_emb, item_emb, ent_emb)
    return _tc_score(u_rows, i_rows, e_rows)


# speedup vs baseline: 1.5723x; 1.5723x over previous
import functools

import jax
import jax.numpy as jnp
from jax import lax
from jax.experimental import pallas as pl
from jax.experimental.pallas import tpu as pltpu
from jax.experimental.pallas import tpu_sc as plsc

B = 16384
D = 64
NC = 2
NS = 16
NW = NC * NS
BPW = B // NW


def _sc_entity_ids(i_ids, item_map):
    mesh = plsc.VectorSubcoreMesh(core_axis_name="c", subcore_axis_name="s")

    @functools.partial(
        pl.kernel,
        mesh=mesh,
        out_type=jax.ShapeDtypeStruct((B,), jnp.int32),
        compiler_params=pltpu.CompilerParams(use_tc_tiling_on_sc=False),
        scratch_types=[
            pltpu.VMEM((BPW,), jnp.int32),
            pltpu.VMEM((BPW,), jnp.int32),
            pltpu.SemaphoreType.DMA,
        ],
    )
    def ka(i_ids_h, map_h, evar_out, iidx, evar_v, s0):
        wid = lax.axis_index("s") * NC + lax.axis_index("c")
        base = wid * BPW
        pltpu.sync_copy(i_ids_h.at[pl.ds(base, BPW)], iidx)
        pltpu.async_copy(map_h.at[iidx], evar_v, s0).wait()
        pltpu.sync_copy(evar_v, evar_out.at[pl.ds(base, BPW)])

    return ka(i_ids, item_map)


def _sc_gather_pairs(u_ids, i_ids, evar, u2, i2, e2):
    mesh = plsc.VectorSubcoreMesh(core_axis_name="c", subcore_axis_name="s")
    out_types = (
        jax.ShapeDtypeStruct((B, 2 * D), jnp.float32),
        jax.ShapeDtypeStruct((B, 2 * D), jnp.float32),
        jax.ShapeDtypeStruct((B, 2 * D), jnp.float32),
    )

    @functools.partial(
        pl.kernel,
        mesh=mesh,
        out_type=out_types,
        scratch_types=[
            pltpu.VMEM((BPW,), jnp.int32),
            pltpu.VMEM((BPW,), jnp.int32),
            pltpu.VMEM((BPW,), jnp.int32),
            pltpu.VMEM((BPW, 2 * D), jnp.float32),
            pltpu.SemaphoreType.DMA,
        ],
    )
    def kb(u_ids_h, i_ids_h, evar_h, u2_h, i2_h, e2_h, u_out, i_out, e_out,
           uh, ih, eh, rows, sem):
        wid = lax.axis_index("s") * NC + lax.axis_index("c")
        base = wid * BPW
        pltpu.sync_copy(u_ids_h.at[pl.ds(base, BPW)], uh)
        pltpu.sync_copy(i_ids_h.at[pl.ds(base, BPW)], ih)
        pltpu.sync_copy(evar_h.at[pl.ds(base, BPW)], eh)

        @pl.loop(0, BPW, step=16)
        def _(c):
            slc = pl.ds(c, 16)
            uh[slc] = uh[slc] >> 1
            ih[slc] = ih[slc] >> 1
            eh[slc] = eh[slc] >> 1

        pltpu.async_copy(u2_h.at[uh], rows, sem).wait()
        pltpu.sync_copy(rows, u_out.at[pl.ds(base, BPW)])
        pltpu.async_copy(i2_h.at[ih], rows, sem).wait()
        pltpu.sync_copy(rows, i_out.at[pl.ds(base, BPW)])
        pltpu.async_copy(e2_h.at[eh], rows, sem).wait()
        pltpu.sync_copy(rows, e_out.at[pl.ds(base, BPW)])

    return kb(u_ids, i_ids, evar, u2, i2, e2)


def _tc_score(u_pair, i_pair, e_pair, u_ids, i_ids, evar):
    def body(u_ref, i_ref, e_ref, uid_ref, iid_ref, eid_ref, o_ref):
        def sel(pair, ids):
            p = (ids & 1)[:, None]
            return jnp.where(p == 1, pair[:, D:], pair[:, :D])

        u = sel(u_ref[...], uid_ref[...])
        ie = (sel(i_ref[...], iid_ref[...]) + sel(e_ref[...], eid_ref[...]))
        o_ref[...] = jnp.sum(u * ie, axis=-1)

    blk = 2048
    grid = B // blk
    pair_spec = pl.BlockSpec((blk, 2 * D), lambda i: (i, 0))
    id_spec = pl.BlockSpec((blk,), lambda i: (i,))
    return pl.pallas_call(
        body,
        grid=(grid,),
        in_specs=[pair_spec, pair_spec, pair_spec, id_spec, id_spec, id_spec],
        out_specs=pl.BlockSpec((blk,), lambda i: (i,)),
        out_shape=jax.ShapeDtypeStruct((B,), jnp.float32),
    )(u_pair, i_pair, e_pair, u_ids, i_ids, evar)


def kernel(u_ids, i_ids, user_emb, item_emb, ent_emb, item_map):
    u_ids = u_ids.astype(jnp.int32)
    i_ids = i_ids.astype(jnp.int32)
    U, De = user_emb.shape
    I, _ = item_emb.shape
    E, _ = ent_emb.shape
    u2 = user_emb.reshape(U // 2, 2 * De)
    i2 = item_emb.reshape(I // 2, 2 * De)
    ent_pad = jnp.pad(ent_emb, ((0, (-E) % 2), (0, 0)))
    e2 = ent_pad.reshape(ent_pad.shape[0] // 2, 2 * De)
    evar = _sc_entity_ids(i_ids, item_map.astype(jnp.int32))
    u_pair, i_pair, e_pair = _sc_gather_pairs(u_ids, i_ids, evar, u2, i2, e2)
    return _tc_score(u_pair, i_pair, e_pair, u_ids, i_ids, evar)


# speedup vs baseline: 1.9726x; 1.2546x over previous
import functools

import jax
import jax.numpy as jnp
from jax import lax
from jax.experimental import pallas as pl
from jax.experimental.pallas import tpu as pltpu
from jax.experimental.pallas import tpu_sc as plsc

B = 16384
D = 64
NC = 2
NS = 16
NW = NC * NS
BPW = B // NW

CB = 1024


def _round_half(v):
    h = (v + 1) // 2
    return ((h + CB - 1) // CB) * CB


def _tc_repack(table_t, half):
    grid = half // CB

    def body(lo_ref, hi_ref, o_ref):
        o_ref[...] = jnp.concatenate([lo_ref[...].T, hi_ref[...].T], axis=1)

    hi_blocks = half // CB
    last_block = (table_t.shape[1] - 1) // CB
    return pl.pallas_call(
        body,
        grid=(grid,),
        in_specs=[
            pl.BlockSpec((D, CB), lambda k: (0, k)),
            pl.BlockSpec((D, CB),
                         lambda k: (0, jnp.minimum(k + hi_blocks, last_block))),
        ],
        out_specs=pl.BlockSpec((CB, 2 * D), lambda k: (k, 0)),
        out_shape=jax.ShapeDtypeStruct((half, 2 * D), jnp.float32),
    )(table_t, table_t)


def _sc_entity_ids(i_ids, item_map):
    mesh = plsc.VectorSubcoreMesh(core_axis_name="c", subcore_axis_name="s")

    @functools.partial(
        pl.kernel,
        mesh=mesh,
        out_type=jax.ShapeDtypeStruct((B,), jnp.int32),
        compiler_params=pltpu.CompilerParams(use_tc_tiling_on_sc=False),
        scratch_types=[
            pltpu.VMEM((BPW,), jnp.int32),
            pltpu.VMEM((BPW,), jnp.int32),
            pltpu.SemaphoreType.DMA,
        ],
    )
    def ka(i_ids_h, map_h, evar_out, iidx, evar_v, s0):
        wid = lax.axis_index("s") * NC + lax.axis_index("c")
        base = wid * BPW
        pltpu.sync_copy(i_ids_h.at[pl.ds(base, BPW)], iidx)
        pltpu.async_copy(map_h.at[iidx], evar_v, s0).wait()
        pltpu.sync_copy(evar_v, evar_out.at[pl.ds(base, BPW)])

    return ka(i_ids, item_map)


def _sc_gather_pairs(u_ids, i_ids, evar, u2, i2, e2, uh_half, ih_half, eh_half):
    mesh = plsc.VectorSubcoreMesh(core_axis_name="c", subcore_axis_name="s")
    out_types = (
        jax.ShapeDtypeStruct((B, 2 * D), jnp.float32),
        jax.ShapeDtypeStruct((B, 2 * D), jnp.float32),
        jax.ShapeDtypeStruct((B, 2 * D), jnp.float32),
    )

    @functools.partial(
        pl.kernel,
        mesh=mesh,
        out_type=out_types,
        scratch_types=[
            pltpu.VMEM((BPW,), jnp.int32),
            pltpu.VMEM((BPW,), jnp.int32),
            pltpu.VMEM((BPW,), jnp.int32),
            pltpu.VMEM((BPW, 2 * D), jnp.float32),
            pltpu.SemaphoreType.DMA,
        ],
    )
    def kb(u_ids_h, i_ids_h, evar_h, u2_h, i2_h, e2_h, u_out, i_out, e_out,
           uh, ih, eh, rows, sem):
        wid = lax.axis_index("s") * NC + lax.axis_index("c")
        base = wid * BPW
        pltpu.sync_copy(u_ids_h.at[pl.ds(base, BPW)], uh)
        pltpu.sync_copy(i_ids_h.at[pl.ds(base, BPW)], ih)
        pltpu.sync_copy(evar_h.at[pl.ds(base, BPW)], eh)

        @pl.loop(0, BPW, step=16)
        def _(c):
            slc = pl.ds(c, 16)
            uv = uh[slc]
            uh[slc] = uv - jnp.where(uv >= uh_half, uh_half, 0)
            iv = ih[slc]
            ih[slc] = iv - jnp.where(iv >= ih_half, ih_half, 0)
            ev = eh[slc]
            eh[slc] = ev - jnp.where(ev >= eh_half, eh_half, 0)

        pltpu.async_copy(u2_h.at[uh], rows, sem).wait()
        pltpu.sync_copy(rows, u_out.at[pl.ds(base, BPW)])
        pltpu.async_copy(i2_h.at[ih], rows, sem).wait()
        pltpu.sync_copy(rows, i_out.at[pl.ds(base, BPW)])
        pltpu.async_copy(e2_h.at[eh], rows, sem).wait()
        pltpu.sync_copy(rows, e_out.at[pl.ds(base, BPW)])

    return kb(u_ids, i_ids, evar, u2, i2, e2)


def _tc_score(u_pair, i_pair, e_pair, u_ids, i_ids, evar, uh, ih, eh):
    def body(u_ref, i_ref, e_ref, uid_ref, iid_ref, eid_ref, o_ref):
        def sel(pair, ids, half):
            return jnp.where(ids >= half, pair[:, D:], pair[:, :D])

        u = sel(u_ref[...], uid_ref[...], uh)
        ie = (sel(i_ref[...], iid_ref[...], ih)
              + sel(e_ref[...], eid_ref[...], eh))
        o_ref[...] = jnp.sum(u * ie, axis=-1)

    blk = 2048
    grid = B // blk
    pair_spec = pl.BlockSpec((blk, 2 * D), lambda i: (i, 0))
    id_spec = pl.BlockSpec((blk, 1), lambda i: (i, 0))
    return pl.pallas_call(
        body,
        grid=(grid,),
        in_specs=[pair_spec, pair_spec, pair_spec, id_spec, id_spec, id_spec],
        out_specs=pl.BlockSpec((blk,), lambda i: (i,)),
        out_shape=jax.ShapeDtypeStruct((B,), jnp.float32),
    )(u_pair, i_pair, e_pair, u_ids.reshape(B, 1), i_ids.reshape(B, 1),
      evar.reshape(B, 1))


def kernel(u_ids, i_ids, user_emb, item_emb, ent_emb, item_map):
    u_ids = u_ids.astype(jnp.int32)
    i_ids = i_ids.astype(jnp.int32)
    U = user_emb.shape[0]
    I = item_emb.shape[0]
    E = ent_emb.shape[0]
    uh, ih, eh = _round_half(U), _round_half(I), _round_half(E)
    u2 = _tc_repack(user_emb.T, uh)
    i2 = _tc_repack(item_emb.T, ih)
    e2 = _tc_repack(ent_emb.T, eh)
    evar = _sc_entity_ids(i_ids, item_map.astype(jnp.int32))
    u_pair, i_pair, e_pair = _sc_gather_pairs(
        u_ids, i_ids, evar, u2, i2, e2, uh, ih, eh)
    return _tc_score(u_pair, i_pair, e_pair, u_ids, i_ids, evar, uh, ih, eh)


# speedup vs baseline: 2.2450x; 1.1381x over previous
import functools

import jax
import jax.numpy as jnp
from jax import lax
from jax.experimental import pallas as pl
from jax.experimental.pallas import tpu as pltpu
from jax.experimental.pallas import tpu_sc as plsc

B = 16384
D = 64
NC = 2
NS = 16
NW = NC * NS
BPW = B // NW

CB = 1024


def _round_half(v):
    h = (v + 1) // 2
    return ((h + CB - 1) // CB) * CB


def _tc_repack(table_t, half):
    grid = half // CB

    def body(lo_ref, hi_ref, o_ref):
        o_ref[...] = jnp.concatenate([lo_ref[...], hi_ref[...]], axis=0).T

    hi_blocks = half // CB
    last_block = (table_t.shape[1] - 1) // CB
    return pl.pallas_call(
        body,
        grid=(grid,),
        in_specs=[
            pl.BlockSpec((D, CB), lambda k: (0, k)),
            pl.BlockSpec((D, CB),
                         lambda k: (0, jnp.minimum(k + hi_blocks, last_block))),
        ],
        out_specs=pl.BlockSpec((CB, 2 * D), lambda k: (k, 0)),
        out_shape=jax.ShapeDtypeStruct((half, 2 * D), jnp.float32),
    )(table_t, table_t)


def _sc_entity_ids(i_ids, item_map):
    mesh = plsc.VectorSubcoreMesh(core_axis_name="c", subcore_axis_name="s")

    @functools.partial(
        pl.kernel,
        mesh=mesh,
        out_type=jax.ShapeDtypeStruct((B,), jnp.int32),
        compiler_params=pltpu.CompilerParams(use_tc_tiling_on_sc=False),
        scratch_types=[
            pltpu.VMEM((BPW,), jnp.int32),
            pltpu.VMEM((BPW,), jnp.int32),
            pltpu.SemaphoreType.DMA,
        ],
    )
    def ka(i_ids_h, map_h, evar_out, iidx, evar_v, s0):
        wid = lax.axis_index("s") * NC + lax.axis_index("c")
        base = wid * BPW
        pltpu.sync_copy(i_ids_h.at[pl.ds(base, BPW)], iidx)
        pltpu.async_copy(map_h.at[iidx], evar_v, s0).wait()
        pltpu.sync_copy(evar_v, evar_out.at[pl.ds(base, BPW)])

    return ka(i_ids, item_map)


def _sc_gather_pairs(u_ids, i_ids, evar, u2, i2, e2, uh_half, ih_half, eh_half):
    mesh = plsc.VectorSubcoreMesh(core_axis_name="c", subcore_axis_name="s")
    out_types = (
        jax.ShapeDtypeStruct((B, 2 * D), jnp.float32),
        jax.ShapeDtypeStruct((B, 2 * D), jnp.float32),
        jax.ShapeDtypeStruct((B, 2 * D), jnp.float32),
    )

    @functools.partial(
        pl.kernel,
        mesh=mesh,
        out_type=out_types,
        scratch_types=[
            pltpu.VMEM((BPW,), jnp.int32),
            pltpu.VMEM((BPW,), jnp.int32),
            pltpu.VMEM((BPW,), jnp.int32),
            pltpu.VMEM((BPW, 2 * D), jnp.float32),
            pltpu.SemaphoreType.DMA,
        ],
    )
    def kb(u_ids_h, i_ids_h, evar_h, u2_h, i2_h, e2_h, u_out, i_out, e_out,
           uh, ih, eh, rows, sem):
        wid = lax.axis_index("s") * NC + lax.axis_index("c")
        base = wid * BPW
        pltpu.sync_copy(u_ids_h.at[pl.ds(base, BPW)], uh)
        pltpu.sync_copy(i_ids_h.at[pl.ds(base, BPW)], ih)
        pltpu.sync_copy(evar_h.at[pl.ds(base, BPW)], eh)

        @pl.loop(0, BPW, step=16)
        def _(c):
            slc = pl.ds(c, 16)
            uv = uh[slc]
            uh[slc] = uv - jnp.where(uv >= uh_half, uh_half, 0)
            iv = ih[slc]
            ih[slc] = iv - jnp.where(iv >= ih_half, ih_half, 0)
            ev = eh[slc]
            eh[slc] = ev - jnp.where(ev >= eh_half, eh_half, 0)

        pltpu.async_copy(u2_h.at[uh], rows, sem).wait()
        pltpu.sync_copy(rows, u_out.at[pl.ds(base, BPW)])
        pltpu.async_copy(i2_h.at[ih], rows, sem).wait()
        pltpu.sync_copy(rows, i_out.at[pl.ds(base, BPW)])
        pltpu.async_copy(e2_h.at[eh], rows, sem).wait()
        pltpu.sync_copy(rows, e_out.at[pl.ds(base, BPW)])

    return kb(u_ids, i_ids, evar, u2, i2, e2)


def _tc_score(u_pair, i_pair, e_pair, u_ids, i_ids, evar, uh, ih, eh):
    def body(u_ref, i_ref, e_ref, uid_ref, iid_ref, eid_ref, o_ref):
        def sel(pair, ids, half):
            return jnp.where(ids >= half, pair[:, D:], pair[:, :D])

        u = sel(u_ref[...], uid_ref[...], uh)
        ie = (sel(i_ref[...], iid_ref[...], ih)
              + sel(e_ref[...], eid_ref[...], eh))
        o_ref[...] = jnp.sum(u * ie, axis=-1)

    blk = 2048
    grid = B // blk
    pair_spec = pl.BlockSpec((blk, 2 * D), lambda i: (i, 0))
    id_spec = pl.BlockSpec((blk, 1), lambda i: (i, 0))
    return pl.pallas_call(
        body,
        grid=(grid,),
        in_specs=[pair_spec, pair_spec, pair_spec, id_spec, id_spec, id_spec],
        out_specs=pl.BlockSpec((blk,), lambda i: (i,)),
        out_shape=jax.ShapeDtypeStruct((B,), jnp.float32),
    )(u_pair, i_pair, e_pair, u_ids.reshape(B, 1), i_ids.reshape(B, 1),
      evar.reshape(B, 1))


def kernel(u_ids, i_ids, user_emb, item_emb, ent_emb, item_map):
    u_ids = u_ids.astype(jnp.int32)
    i_ids = i_ids.astype(jnp.int32)
    U = user_emb.shape[0]
    I = item_emb.shape[0]
    E = ent_emb.shape[0]
    uh, ih, eh = _round_half(U), _round_half(I), _round_half(E)
    u2 = _tc_repack(user_emb.T, uh)
    i2 = _tc_repack(item_emb.T, ih)
    e2 = _tc_repack(ent_emb.T, eh)
    evar = _sc_entity_ids(i_ids, item_map.astype(jnp.int32))
    u_pair, i_pair, e_pair = _sc_gather_pairs(
        u_ids, i_ids, evar, u2, i2, e2, uh, ih, eh)
    return _tc_score(u_pair, i_pair, e_pair, u_ids, i_ids, evar, uh, ih, eh)
